# baseline (speedup 1.0000x reference)
import jax
import jax.numpy as jnp
from jax.experimental import pallas as pl
import jax.experimental.pallas.tpu as pltpu

N = 8192
I_BLK = 256
J_BLK = 2048
LANES = 128
CHUNKS = J_BLK // LANES
HALF = J_BLK // 2
BIG = 3.0e38


def _prep_kernel(pc_ref, pcT_ref, sqi_ref, sqj_ref):
    xc = pc_ref[:, 0:1]
    yc = pc_ref[:, 1:2]
    sq_col = xc * xc + yc * yc
    sqi_ref[...] = jnp.broadcast_to(sq_col, (N, LANES))
    xr = pcT_ref[0:1, :]
    yr = pcT_ref[1:2, :]
    sq_row = xr * xr + yr * yr
    sqj_ref[...] = jnp.broadcast_to(sq_row, (I_BLK, N))


def _nn_kernel(a_ref, b_ref, sqi_ref, sqj_ref, esdf_ref, idx_ref,
               m2a_ref, m2b_ref, acc_val, acc_idx):
    j = pl.program_id(0)
    i = pl.program_id(1)
    isl = pl.ds(i * I_BLK, I_BLK)

    @pl.when(j == 0)
    def _init():
        acc_val[isl, :] = jnp.full((I_BLK, LANES), BIG, jnp.float32)
        acc_idx[isl, :] = jnp.zeros((I_BLK, LANES), jnp.int32)

    a2 = a_ref[...] * 2.0
    m2a_ref[...] = jnp.dot(a2, b_ref[:, :HALF],
                           preferred_element_type=jnp.float32)
    m2b_ref[...] = jnp.dot(a2, b_ref[:, HALF:],
                           preferred_element_type=jnp.float32)

    @pl.when((i // (J_BLK // I_BLK)) == j)
    def _patch_diag():
        off = i * I_BLK - j * J_BLK
        rr = jax.lax.broadcasted_iota(jnp.int32, (I_BLK, I_BLK), 0)
        cc = jax.lax.broadcasted_iota(jnp.int32, (I_BLK, I_BLK), 1)
        eye = rr == cc

        @pl.when(off < HALF)
        def _pa():
            sl2 = pl.ds(off, I_BLK)
            m2a_ref[:, sl2] = jnp.where(eye, -BIG, m2a_ref[:, sl2])

        @pl.when(off >= HALF)
        def _pb():
            sl2 = pl.ds(off - HALF, I_BLK)
            m2b_ref[:, sl2] = jnp.where(eye, -BIG, m2b_ref[:, sl2])

    sqi = sqi_ref[...]
    tile_v = None
    tile_c = None
    for c in range(CHUNKS):
        sl = slice(c * LANES, (c + 1) * LANES)
        hl = slice((c * LANES) % HALF, (c * LANES) % HALF + LANES)
        m2c = m2a_ref[:, hl] if c < CHUNKS // 2 else m2b_ref[:, hl]
        t = sqi + sqj_ref[:, sl]
        d2 = jnp.maximum(t - m2c, jnp.float32(1e-12))
        if c == 0:
            tile_v = d2
            tile_c = jnp.zeros((I_BLK, LANES), jnp.int32)
        else:
            better = d2 < tile_v
            tile_c = jnp.where(better, jnp.int32(c), tile_c)
            tile_v = jnp.minimum(tile_v, d2)

    lane = jax.lax.broadcasted_iota(jnp.int32, (I_BLK, LANES), 1)
    tile_idx = (tile_c * LANES) + (lane + j * J_BLK)
    better = tile_v < acc_val[isl, :]
    acc_idx[isl, :] = jnp.where(better, tile_idx, acc_idx[isl, :])
    acc_val[isl, :] = jnp.minimum(acc_val[isl, :], tile_v)

    @pl.when(j == (N // J_BLK) - 1)
    def _finish():
        av = acc_val[isl, :]
        ai = acc_idx[isl, :]
        rmin = jnp.min(av, axis=1)
        cand = jnp.where(av == rmin[:, None], ai, jnp.int32(2**30))
        ridx = jnp.min(cand, axis=1)
        esdf_ref[isl] = jnp.sqrt(rmin)
        idx_ref[isl] = ridx


@jax.jit
def _nn_argmin(point_cloud):
    pcT = point_cloud.T
    sqi_b, sqj_b = pl.pallas_call(
        _prep_kernel,
        out_shape=[
            jax.ShapeDtypeStruct((N, LANES), jnp.float32),
            jax.ShapeDtypeStruct((I_BLK, N), jnp.float32),
        ],
    )(point_cloud, pcT)
    esdf, idx = pl.pallas_call(
        _nn_kernel,
        grid=(N // J_BLK, N // I_BLK),
        in_specs=[
            pl.BlockSpec((I_BLK, 2), lambda j, i: (i, 0)),
            pl.BlockSpec((2, J_BLK), lambda j, i: (0, j)),
            pl.BlockSpec((I_BLK, LANES), lambda j, i: (i, 0)),
            pl.BlockSpec((I_BLK, J_BLK), lambda j, i: (0, j)),
        ],
        out_specs=[
            pl.BlockSpec((N,), lambda j, i: (0,)),
            pl.BlockSpec((N,), lambda j, i: (0,)),
        ],
        out_shape=[
            jax.ShapeDtypeStruct((N,), jnp.float32),
            jax.ShapeDtypeStruct((N,), jnp.int32),
        ],
        scratch_shapes=[
            pltpu.VMEM((I_BLK, HALF), jnp.float32),
            pltpu.VMEM((I_BLK, HALF), jnp.float32),
            pltpu.VMEM((N, LANES), jnp.float32),
            pltpu.VMEM((N, LANES), jnp.int32),
        ],
    )(point_cloud, pcT, sqi_b, sqj_b)
    return esdf, idx


def kernel(point_cloud):
    esdf, idx = _nn_argmin(point_cloud)
    nearest = point_cloud[idx]
    direction = point_cloud - nearest
    norm = jnp.linalg.norm(direction, axis=1, keepdims=True)
    gradients = direction / (norm + 1e-8)
    gx = gradients[:, 0]
    gy = gradients[:, 1]
    mu = jnp.stack([gx, -gx, gy, -gy], axis=0)
    lam = jnp.stack([gx, gy, esdf / 10.0], axis=0)
    return (mu, lam)


# speedup vs baseline: 1.1335x; 1.1335x over previous
import functools

import jax
import jax.numpy as jnp
from jax import lax
from jax.experimental import pallas as pl
import jax.experimental.pallas.tpu as pltpu
from jax.experimental.pallas import tpu_sc as plsc

N = 8192
I_BLK = 256
J_BLK = 2048
LANES = 128
CHUNKS = J_BLK // LANES
HALF = J_BLK // 2
BIG = 3.0e38
EROWS = 2 * N + I_BLK


def _prep_kernel(pc_ref, pcT_ref, sqi_ref, sqj_ref, bias_ref):
    xc = pc_ref[:, 0:1]
    yc = pc_ref[:, 1:2]
    sq_col = xc * xc + yc * yc
    sqi_ref[...] = jnp.broadcast_to(sq_col, (N, LANES))
    xr = pcT_ref[0:1, :]
    yr = pcT_ref[1:2, :]
    sq_row = xr * xr + yr * yr
    sqj_ref[...] = jnp.broadcast_to(sq_row, (I_BLK, N))
    rk = jax.lax.broadcasted_iota(jnp.int32, (EROWS, LANES), 0)
    lk = jax.lax.broadcasted_iota(jnp.int32, (EROWS, LANES), 1)
    bias_ref[...] = jnp.where(rk - N == lk, BIG, jnp.float32(1e-12))


def _nn_kernel(a_ref, b_ref, sqi_ref, sqj_ref, bias_ref, esdf_ref, idx_ref,
               m2a_ref, m2b_ref, acc_val, acc_idx):
    j = pl.program_id(0)
    i = pl.program_id(1)
    isl = pl.ds(i * I_BLK, I_BLK)

    @pl.when(j == 0)
    def _init():
        acc_val[isl, :] = jnp.full((I_BLK, LANES), BIG, jnp.float32)
        acc_idx[isl, :] = jnp.zeros((I_BLK, LANES), jnp.int32)

    a2 = a_ref[...] * 2.0
    m2a_ref[...] = jnp.dot(a2, b_ref[:, :HALF],
                           preferred_element_type=jnp.float32)
    m2b_ref[...] = jnp.dot(a2, b_ref[:, HALF:],
                           preferred_element_type=jnp.float32)

    sqi = sqi_ref[...]
    tile_v = None
    tile_c = None
    for c in range(CHUNKS):
        sl = slice(c * LANES, (c + 1) * LANES)
        hl = slice((c * LANES) % HALF, (c * LANES) % HALF + LANES)
        m2c = m2a_ref[:, hl] if c < CHUNKS // 2 else m2b_ref[:, hl]
        t = sqi + sqj_ref[:, sl]
        bias = bias_ref[pl.ds(N + i * I_BLK - j * J_BLK - c * LANES, I_BLK), :]
        d2 = jnp.maximum(t - m2c, bias)
        if c == 0:
            tile_v = d2
            tile_c = jnp.zeros((I_BLK, LANES), jnp.int32)
        else:
            better = d2 < tile_v
            tile_c = jnp.where(better, jnp.int32(c), tile_c)
            tile_v = jnp.minimum(tile_v, d2)

    lane = jax.lax.broadcasted_iota(jnp.int32, (I_BLK, LANES), 1)
    tile_idx = (tile_c * LANES) + (lane + j * J_BLK)
    better = tile_v < acc_val[isl, :]
    acc_idx[isl, :] = jnp.where(better, tile_idx, acc_idx[isl, :])
    acc_val[isl, :] = jnp.minimum(acc_val[isl, :], tile_v)

    @pl.when(j == (N // J_BLK) - 1)
    def _finish():
        av = acc_val[isl, :]
        ai = acc_idx[isl, :]
        rmin = jnp.min(av, axis=1)
        cand = jnp.where(av == rmin[:, None], ai, jnp.int32(2**30))
        ridx = jnp.min(cand, axis=1)
        esdf_ref[isl] = jnp.sqrt(rmin)
        idx_ref[isl] = ridx


@jax.jit
def _nn_argmin(point_cloud):
    pcT = point_cloud.T
    sqi_b, sqj_b, bias = pl.pallas_call(
        _prep_kernel,
        out_shape=[
            jax.ShapeDtypeStruct((N, LANES), jnp.float32),
            jax.ShapeDtypeStruct((I_BLK, N), jnp.float32),
            jax.ShapeDtypeStruct((EROWS, LANES), jnp.float32),
        ],
    )(point_cloud, pcT)
    esdf, idx = pl.pallas_call(
        _nn_kernel,
        grid=(N // J_BLK, N // I_BLK),
        in_specs=[
            pl.BlockSpec((I_BLK, 2), lambda j, i: (i, 0)),
            pl.BlockSpec((2, J_BLK), lambda j, i: (0, j)),
            pl.BlockSpec((I_BLK, LANES), lambda j, i: (i, 0)),
            pl.BlockSpec((I_BLK, J_BLK), lambda j, i: (0, j)),
            pl.BlockSpec((EROWS, LANES), lambda j, i: (0, 0)),
        ],
        out_specs=[
            pl.BlockSpec((N,), lambda j, i: (0,)),
            pl.BlockSpec((N,), lambda j, i: (0,)),
        ],
        out_shape=[
            jax.ShapeDtypeStruct((N,), jnp.float32),
            jax.ShapeDtypeStruct((N,), jnp.int32),
        ],
        scratch_shapes=[
            pltpu.VMEM((I_BLK, HALF), jnp.float32),
            pltpu.VMEM((I_BLK, HALF), jnp.float32),
            pltpu.VMEM((N, LANES), jnp.float32),
            pltpu.VMEM((N, LANES), jnp.int32),
        ],
    )(point_cloud, pcT, sqi_b, sqj_b, bias)
    return esdf, idx


_SC_WORKERS = 32
_PTS_PER_W = N // _SC_WORKERS
_VW = 16


def _grad_sc_kernel(x_hbm, y_hbm, idx_hbm, gx_hbm, gy_hbm,
                    mx_v, my_v, nx_v, ny_v, idx_v, gx_v, gy_v, sem):
    wid = lax.axis_index("s") * 2 + lax.axis_index("c")
    base = wid * _PTS_PER_W
    bsl = pl.ds(base, _PTS_PER_W)
    pltpu.sync_copy(idx_hbm.at[bsl], idx_v)
    pltpu.async_copy(x_hbm.at[idx_v], nx_v, sem).wait()
    pltpu.async_copy(y_hbm.at[idx_v], ny_v, sem).wait()
    pltpu.sync_copy(x_hbm.at[bsl], mx_v)
    pltpu.sync_copy(y_hbm.at[bsl], my_v)
    for k in range(_PTS_PER_W // _VW):
        sl = pl.ds(k * _VW, _VW)
        nx = nx_v[sl]
        ny = ny_v[sl]
        mx = mx_v[sl]
        my = my_v[sl]
        dx = mx - nx
        dy = my - ny
        d = dx * dx + dy * dy
        ib = lax.bitcast_convert_type(d, jnp.int32)
        seed = jnp.int32(0x5F3759DF) - (ib >> 1)
        r = lax.bitcast_convert_type(seed, jnp.float32)
        for _ in range(3):
            r = r * (1.5 - 0.5 * d * r * r)
        nrm = d * r
        a = nrm + 1e-8
        w = r
        for _ in range(2):
            w = w * (2.0 - a * w)
        w = jnp.where(d > 0.0, w, 0.0)
        gx_v[sl] = dx * w
        gy_v[sl] = dy * w
    pltpu.sync_copy(gx_v, gx_hbm.at[pl.ds(base, _PTS_PER_W)])
    pltpu.sync_copy(gy_v, gy_hbm.at[pl.ds(base, _PTS_PER_W)])


@jax.jit
def _nn_gradients(x, y, idx):
    mesh = plsc.VectorSubcoreMesh(core_axis_name="c", subcore_axis_name="s")
    run = functools.partial(
        pl.kernel,
        mesh=mesh,
        out_type=[
            jax.ShapeDtypeStruct((N,), jnp.float32),
            jax.ShapeDtypeStruct((N,), jnp.float32),
        ],
        scratch_types=[
            pltpu.VMEM((_PTS_PER_W,), jnp.float32),
            pltpu.VMEM((_PTS_PER_W,), jnp.float32),
            pltpu.VMEM((_PTS_PER_W,), jnp.float32),
            pltpu.VMEM((_PTS_PER_W,), jnp.float32),
            pltpu.VMEM((_PTS_PER_W,), jnp.int32),
            pltpu.VMEM((_PTS_PER_W,), jnp.float32),
            pltpu.VMEM((_PTS_PER_W,), jnp.float32),
            pltpu.SemaphoreType.DMA,
        ],
    )(_grad_sc_kernel)
    return run(x, y, idx)


def kernel(point_cloud):
    esdf, idx = _nn_argmin(point_cloud)
    gx, gy = _nn_gradients(point_cloud[:, 0], point_cloud[:, 1], idx)
    mu = jnp.stack([gx, -gx, gy, -gy], axis=0)
    lam = jnp.stack([gx, gy, esdf / 10.0], axis=0)
    return (mu, lam)


# speedup vs baseline: 1.3332x; 1.1762x over previous
import functools

import jax
import jax.numpy as jnp
from jax import lax
from jax.experimental import pallas as pl
import jax.experimental.pallas.tpu as pltpu
from jax.experimental.pallas import tpu_sc as plsc

N = 8192
I_BLK = 256
J_BLK = 4096
LANES = 128
CHUNKS = J_BLK // LANES
HALF = J_BLK // 2
BIG = 3.0e38
EROWS = 2 * N + I_BLK


def _prep_kernel(pc_ref, pcT_ref, sqi_ref, sqj_ref, bias_ref):
    xc = pc_ref[:, 0:1]
    yc = pc_ref[:, 1:2]
    sq_col = xc * xc + yc * yc
    sqi_ref[...] = jnp.broadcast_to(sq_col, (N, LANES))
    xr = pcT_ref[0:1, :]
    yr = pcT_ref[1:2, :]
    sq_row = xr * xr + yr * yr
    sqj_ref[...] = jnp.broadcast_to(sq_row, (I_BLK, N))
    rk = jax.lax.broadcasted_iota(jnp.int32, (EROWS, LANES), 0)
    lk = jax.lax.broadcasted_iota(jnp.int32, (EROWS, LANES), 1)
    bias_ref[...] = jnp.where(rk - N == lk, BIG, jnp.float32(1e-12))


def _nn_kernel(a_ref, b_ref, sqi_ref, sqj_ref, bias_ref, esdf_ref, idx_ref,
               m2a_ref, m2b_ref, acc_val, acc_idx):
    j = pl.program_id(0)
    i = pl.program_id(1)
    isl = pl.ds(i * I_BLK, I_BLK)

    @pl.when(j == 0)
    def _init():
        acc_val[isl, :] = jnp.full((I_BLK, LANES), BIG, jnp.float32)
        acc_idx[isl, :] = jnp.zeros((I_BLK, LANES), jnp.int32)

    a2 = a_ref[...] * 2.0
    m2a_ref[...] = jnp.dot(a2, b_ref[:, :HALF],
                           preferred_element_type=jnp.float32)
    m2b_ref[...] = jnp.dot(a2, b_ref[:, HALF:],
                           preferred_element_type=jnp.float32)

    sqi = sqi_ref[...]
    tile_v = None
    tile_c = None
    for c in range(CHUNKS):
        sl = slice(c * LANES, (c + 1) * LANES)
        hl = slice((c * LANES) % HALF, (c * LANES) % HALF + LANES)
        m2c = m2a_ref[:, hl] if c < CHUNKS // 2 else m2b_ref[:, hl]
        t = sqi + sqj_ref[:, sl]
        bias = bias_ref[pl.ds(N + i * I_BLK - j * J_BLK - c * LANES, I_BLK), :]
        d2 = jnp.maximum(t - m2c, bias)
        if c == 0:
            tile_v = d2
            tile_c = jnp.zeros((I_BLK, LANES), jnp.int32)
        else:
            better = d2 < tile_v
            tile_c = jnp.where(better, jnp.int32(c), tile_c)
            tile_v = jnp.minimum(tile_v, d2)

    lane = jax.lax.broadcasted_iota(jnp.int32, (I_BLK, LANES), 1)
    tile_idx = (tile_c * LANES) + (lane + j * J_BLK)
    better = tile_v < acc_val[isl, :]
    acc_idx[isl, :] = jnp.where(better, tile_idx, acc_idx[isl, :])
    acc_val[isl, :] = jnp.minimum(acc_val[isl, :], tile_v)

    @pl.when(j == (N // J_BLK) - 1)
    def _finish():
        av = acc_val[isl, :]
        ai = acc_idx[isl, :]
        rmin = jnp.min(av, axis=1)
        cand = jnp.where(av == rmin[:, None], ai, jnp.int32(2**30))
        ridx = jnp.min(cand, axis=1)
        esdf_ref[isl] = jnp.sqrt(rmin)
        idx_ref[isl] = ridx


@jax.jit
def _nn_argmin(point_cloud):
    pcT = point_cloud.T
    sqi_b, sqj_b, bias = pl.pallas_call(
        _prep_kernel,
        out_shape=[
            jax.ShapeDtypeStruct((N, LANES), jnp.float32),
            jax.ShapeDtypeStruct((I_BLK, N), jnp.float32),
            jax.ShapeDtypeStruct((EROWS, LANES), jnp.float32),
        ],
    )(point_cloud, pcT)
    esdf, idx = pl.pallas_call(
        _nn_kernel,
        grid=(N // J_BLK, N // I_BLK),
        in_specs=[
            pl.BlockSpec((I_BLK, 2), lambda j, i: (i, 0)),
            pl.BlockSpec((2, J_BLK), lambda j, i: (0, j)),
            pl.BlockSpec((I_BLK, LANES), lambda j, i: (i, 0)),
            pl.BlockSpec((I_BLK, J_BLK), lambda j, i: (0, j)),
            pl.BlockSpec((EROWS, LANES), lambda j, i: (0, 0)),
        ],
        out_specs=[
            pl.BlockSpec((N,), lambda j, i: (0,)),
            pl.BlockSpec((N,), lambda j, i: (0,)),
        ],
        out_shape=[
            jax.ShapeDtypeStruct((N,), jnp.float32),
            jax.ShapeDtypeStruct((N,), jnp.int32),
        ],
        scratch_shapes=[
            pltpu.VMEM((I_BLK, HALF), jnp.float32),
            pltpu.VMEM((I_BLK, HALF), jnp.float32),
            pltpu.VMEM((N, LANES), jnp.float32),
            pltpu.VMEM((N, LANES), jnp.int32),
        ],
    )(point_cloud, pcT, sqi_b, sqj_b, bias)
    return esdf, idx


_SC_WORKERS = 32
_PTS_PER_W = N // _SC_WORKERS
_VW = 16


def _grad_sc_kernel(x_hbm, y_hbm, idx_hbm, gx_hbm, gy_hbm,
                    mx_v, my_v, nx_v, ny_v, idx_v, gx_v, gy_v, sem):
    wid = lax.axis_index("s") * 2 + lax.axis_index("c")
    base = wid * _PTS_PER_W
    bsl = pl.ds(base, _PTS_PER_W)
    pltpu.sync_copy(idx_hbm.at[bsl], idx_v)
    pltpu.async_copy(x_hbm.at[idx_v], nx_v, sem).wait()
    pltpu.async_copy(y_hbm.at[idx_v], ny_v, sem).wait()
    pltpu.sync_copy(x_hbm.at[bsl], mx_v)
    pltpu.sync_copy(y_hbm.at[bsl], my_v)
    for k in range(_PTS_PER_W // _VW):
        sl = pl.ds(k * _VW, _VW)
        nx = nx_v[sl]
        ny = ny_v[sl]
        mx = mx_v[sl]
        my = my_v[sl]
        dx = mx - nx
        dy = my - ny
        d = dx * dx + dy * dy
        ib = lax.bitcast_convert_type(d, jnp.int32)
        seed = jnp.int32(0x5F3759DF) - (ib >> 1)
        r = lax.bitcast_convert_type(seed, jnp.float32)
        for _ in range(3):
            r = r * (1.5 - 0.5 * d * r * r)
        nrm = d * r
        a = nrm + 1e-8
        w = r
        for _ in range(2):
            w = w * (2.0 - a * w)
        w = jnp.where(d > 0.0, w, 0.0)
        gx_v[sl] = dx * w
        gy_v[sl] = dy * w
    pltpu.sync_copy(gx_v, gx_hbm.at[pl.ds(base, _PTS_PER_W)])
    pltpu.sync_copy(gy_v, gy_hbm.at[pl.ds(base, _PTS_PER_W)])


@jax.jit
def _nn_gradients(x, y, idx):
    mesh = plsc.VectorSubcoreMesh(core_axis_name="c", subcore_axis_name="s")
    run = functools.partial(
        pl.kernel,
        mesh=mesh,
        out_type=[
            jax.ShapeDtypeStruct((N,), jnp.float32),
            jax.ShapeDtypeStruct((N,), jnp.float32),
        ],
        scratch_types=[
            pltpu.VMEM((_PTS_PER_W,), jnp.float32),
            pltpu.VMEM((_PTS_PER_W,), jnp.float32),
            pltpu.VMEM((_PTS_PER_W,), jnp.float32),
            pltpu.VMEM((_PTS_PER_W,), jnp.float32),
            pltpu.VMEM((_PTS_PER_W,), jnp.int32),
            pltpu.VMEM((_PTS_PER_W,), jnp.float32),
            pltpu.VMEM((_PTS_PER_W,), jnp.float32),
            pltpu.SemaphoreType.DMA,
        ],
    )(_grad_sc_kernel)
    return run(x, y, idx)


def kernel(point_cloud):
    esdf, idx = _nn_argmin(point_cloud)
    gx, gy = _nn_gradients(point_cloud[:, 0], point_cloud[:, 1], idx)
    mu = jnp.stack([gx, -gx, gy, -gy], axis=0)
    lam = jnp.stack([gx, gy, esdf / 10.0], axis=0)
    return (mu, lam)


# speedup vs baseline: 1.4046x; 1.0536x over previous
import functools

import jax
import jax.numpy as jnp
from jax import lax
from jax.experimental import pallas as pl
import jax.experimental.pallas.tpu as pltpu
from jax.experimental.pallas import tpu_sc as plsc

N = 8192
I_BLK = 256
J_BLK = 8192
LANES = 128
CHUNKS = J_BLK // LANES
HALF = J_BLK // 2
BIG = 3.0e38
EROWS = 2 * N + I_BLK


def _prep_kernel(pc_ref, pcT_ref, sqi_ref, sqj_ref, bias_ref):
    xc = pc_ref[:, 0:1]
    yc = pc_ref[:, 1:2]
    sq_col = xc * xc + yc * yc
    sqi_ref[...] = jnp.broadcast_to(sq_col, (N, LANES))
    xr = pcT_ref[0:1, :]
    yr = pcT_ref[1:2, :]
    sq_row = xr * xr + yr * yr
    sqj_ref[...] = jnp.broadcast_to(sq_row, (I_BLK, N))
    rk = jax.lax.broadcasted_iota(jnp.int32, (EROWS, LANES), 0)
    lk = jax.lax.broadcasted_iota(jnp.int32, (EROWS, LANES), 1)
    bias_ref[...] = jnp.where(rk - N == lk, BIG, jnp.float32(1e-12))


def _nn_kernel(a_ref, b_ref, sqi_ref, sqj_ref, bias_ref, esdf_ref, idx_ref,
               m2a_ref, m2b_ref, acc_val, acc_idx):
    j = pl.program_id(0)
    i = pl.program_id(1)
    isl = pl.ds(i * I_BLK, I_BLK)

    @pl.when(j == 0)
    def _init():
        acc_val[isl, :] = jnp.full((I_BLK, LANES), BIG, jnp.float32)
        acc_idx[isl, :] = jnp.zeros((I_BLK, LANES), jnp.int32)

    a2 = a_ref[...] * 2.0
    m2a_ref[...] = jnp.dot(a2, b_ref[:, :HALF],
                           preferred_element_type=jnp.float32)
    m2b_ref[...] = jnp.dot(a2, b_ref[:, HALF:],
                           preferred_element_type=jnp.float32)

    sqi = sqi_ref[...]
    tile_v = None
    tile_c = None
    for c in range(CHUNKS):
        sl = slice(c * LANES, (c + 1) * LANES)
        hl = slice((c * LANES) % HALF, (c * LANES) % HALF + LANES)
        m2c = m2a_ref[:, hl] if c < CHUNKS // 2 else m2b_ref[:, hl]
        t = sqi + sqj_ref[:, sl]
        bias = bias_ref[pl.ds(N + i * I_BLK - j * J_BLK - c * LANES, I_BLK), :]
        d2 = jnp.maximum(t - m2c, bias)
        if c == 0:
            tile_v = d2
            tile_c = jnp.zeros((I_BLK, LANES), jnp.int32)
        else:
            better = d2 < tile_v
            tile_c = jnp.where(better, jnp.int32(c), tile_c)
            tile_v = jnp.minimum(tile_v, d2)

    lane = jax.lax.broadcasted_iota(jnp.int32, (I_BLK, LANES), 1)
    tile_idx = (tile_c * LANES) + (lane + j * J_BLK)
    better = tile_v < acc_val[isl, :]
    acc_idx[isl, :] = jnp.where(better, tile_idx, acc_idx[isl, :])
    acc_val[isl, :] = jnp.minimum(acc_val[isl, :], tile_v)

    @pl.when(j == (N // J_BLK) - 1)
    def _finish():
        av = acc_val[isl, :]
        ai = acc_idx[isl, :]
        rmin = jnp.min(av, axis=1)
        cand = jnp.where(av == rmin[:, None], ai, jnp.int32(2**30))
        ridx = jnp.min(cand, axis=1)
        esdf_ref[isl] = jnp.sqrt(rmin)
        idx_ref[isl] = ridx


@jax.jit
def _nn_argmin(point_cloud):
    pcT = point_cloud.T
    sqi_b, sqj_b, bias = pl.pallas_call(
        _prep_kernel,
        out_shape=[
            jax.ShapeDtypeStruct((N, LANES), jnp.float32),
            jax.ShapeDtypeStruct((I_BLK, N), jnp.float32),
            jax.ShapeDtypeStruct((EROWS, LANES), jnp.float32),
        ],
    )(point_cloud, pcT)
    esdf, idx = pl.pallas_call(
        _nn_kernel,
        grid=(N // J_BLK, N // I_BLK),
        in_specs=[
            pl.BlockSpec((I_BLK, 2), lambda j, i: (i, 0)),
            pl.BlockSpec((2, J_BLK), lambda j, i: (0, j)),
            pl.BlockSpec((I_BLK, LANES), lambda j, i: (i, 0)),
            pl.BlockSpec((I_BLK, J_BLK), lambda j, i: (0, j)),
            pl.BlockSpec((EROWS, LANES), lambda j, i: (0, 0)),
        ],
        out_specs=[
            pl.BlockSpec((N,), lambda j, i: (0,)),
            pl.BlockSpec((N,), lambda j, i: (0,)),
        ],
        out_shape=[
            jax.ShapeDtypeStruct((N,), jnp.float32),
            jax.ShapeDtypeStruct((N,), jnp.int32),
        ],
        scratch_shapes=[
            pltpu.VMEM((I_BLK, HALF), jnp.float32),
            pltpu.VMEM((I_BLK, HALF), jnp.float32),
            pltpu.VMEM((N, LANES), jnp.float32),
            pltpu.VMEM((N, LANES), jnp.int32),
        ],
    )(point_cloud, pcT, sqi_b, sqj_b, bias)
    return esdf, idx


_SC_WORKERS = 32
_PTS_PER_W = N // _SC_WORKERS
_VW = 16


def _grad_sc_kernel(x_hbm, y_hbm, idx_hbm, gx_hbm, gy_hbm,
                    mx_v, my_v, nx_v, ny_v, idx_v, gx_v, gy_v, sem):
    wid = lax.axis_index("s") * 2 + lax.axis_index("c")
    base = wid * _PTS_PER_W
    bsl = pl.ds(base, _PTS_PER_W)
    pltpu.sync_copy(idx_hbm.at[bsl], idx_v)
    pltpu.async_copy(x_hbm.at[idx_v], nx_v, sem).wait()
    pltpu.async_copy(y_hbm.at[idx_v], ny_v, sem).wait()
    pltpu.sync_copy(x_hbm.at[bsl], mx_v)
    pltpu.sync_copy(y_hbm.at[bsl], my_v)
    for k in range(_PTS_PER_W // _VW):
        sl = pl.ds(k * _VW, _VW)
        nx = nx_v[sl]
        ny = ny_v[sl]
        mx = mx_v[sl]
        my = my_v[sl]
        dx = mx - nx
        dy = my - ny
        d = dx * dx + dy * dy
        ib = lax.bitcast_convert_type(d, jnp.int32)
        seed = jnp.int32(0x5F3759DF) - (ib >> 1)
        r = lax.bitcast_convert_type(seed, jnp.float32)
        for _ in range(3):
            r = r * (1.5 - 0.5 * d * r * r)
        nrm = d * r
        a = nrm + 1e-8
        w = r
        for _ in range(2):
            w = w * (2.0 - a * w)
        w = jnp.where(d > 0.0, w, 0.0)
        gx_v[sl] = dx * w
        gy_v[sl] = dy * w
    pltpu.sync_copy(gx_v, gx_hbm.at[pl.ds(base, _PTS_PER_W)])
    pltpu.sync_copy(gy_v, gy_hbm.at[pl.ds(base, _PTS_PER_W)])


@jax.jit
def _nn_gradients(x, y, idx):
    mesh = plsc.VectorSubcoreMesh(core_axis_name="c", subcore_axis_name="s")
    run = functools.partial(
        pl.kernel,
        mesh=mesh,
        out_type=[
            jax.ShapeDtypeStruct((N,), jnp.float32),
            jax.ShapeDtypeStruct((N,), jnp.float32),
        ],
        scratch_types=[
            pltpu.VMEM((_PTS_PER_W,), jnp.float32),
            pltpu.VMEM((_PTS_PER_W,), jnp.float32),
            pltpu.VMEM((_PTS_PER_W,), jnp.float32),
            pltpu.VMEM((_PTS_PER_W,), jnp.float32),
            pltpu.VMEM((_PTS_PER_W,), jnp.int32),
            pltpu.VMEM((_PTS_PER_W,), jnp.float32),
            pltpu.VMEM((_PTS_PER_W,), jnp.float32),
            pltpu.SemaphoreType.DMA,
        ],
    )(_grad_sc_kernel)
    return run(x, y, idx)


def kernel(point_cloud):
    esdf, idx = _nn_argmin(point_cloud)
    gx, gy = _nn_gradients(point_cloud[:, 0], point_cloud[:, 1], idx)
    mu = jnp.stack([gx, -gx, gy, -gy], axis=0)
    lam = jnp.stack([gx, gy, esdf / 10.0], axis=0)
    return (mu, lam)
